# baseline (speedup 1.0000x reference)
import functools

import jax
import jax.numpy as jnp
from jax import lax
from jax.experimental import pallas as pl
from jax.experimental.pallas import tpu as pltpu
from jax.experimental.pallas import tpu_sc as plsc

BATCH = 16384
NE = 1000000
NR = 1000
D = 64
DP = 128
L = 16
NC = 2
NS = 16
NW = NC * NS
CHUNK = BATCH // NW
W = 128
NSUB = CHUNK // W

BN = 1024
NBLK = pl.cdiv(NE, BN)
NOUT = pl.cdiv(NE, 8 * BN)
NE2 = NOUT * 4 * BN


def _xpose_body(e_ref, e2_ref):
    ident = (lax.broadcasted_iota(jnp.int32, (4 * D, 4 * D), 0)
             == lax.broadcasted_iota(jnp.int32, (4 * D, 4 * D), 1)
             ).astype(jnp.float32)
    dn = (((0,), (0,)), ((), ()))
    hi = lax.Precision.DEFAULT

    def xpose4(x):
        x4 = jnp.concatenate(
            [x[:, kk * BN:(kk + 1) * BN] for kk in range(4)], axis=0)
        return lax.dot_general(x4, ident, dn,
                               preferred_element_type=jnp.float32,
                               precision=hi)

    xe = e_ref[...]
    for h in range(2):
        ye = xpose4(xe[:, 4 * h * BN:(4 * h + 4) * BN])
        for j in range(4):
            k = 4 * h + j
            half = (k & 1) * D
            row0 = (k >> 1) * BN
            e2_ref[row0:row0 + BN, half:half + D] = ye[:, j * D:(j + 1) * D]


def _transpose_table(T):
    in_spec = pl.BlockSpec((D, 8 * BN), lambda i: (0, i))
    out_spec = pl.BlockSpec((4 * BN, DP), lambda i: (i, 0))
    return pl.pallas_call(
        _xpose_body,
        grid=(NOUT,),
        in_specs=[in_spec],
        out_specs=out_spec,
        out_shape=jax.ShapeDtypeStruct((NE2, DP), jnp.float32),
        compiler_params=pltpu.CompilerParams(
            fuse_transposed_lhs_in_matmul=True),
    )(T.T)


def _prep_indices(s_v, r_v, o_v, sp_v, rp_v, op_v):
    def split(g, carry):
        sl = pl.ds(g * L, L)
        sv = s_v[sl]
        rv = r_v[sl]
        ov = o_v[sl]
        sp_v[sl] = (lax.shift_right_logical(sv, 10) & 1) * D
        op_v[sl] = (lax.shift_right_logical(ov, 10) & 1) * D
        rp_v[sl] = (rv & 1) * D
        s_v[sl] = ((lax.shift_right_logical(sv, 11) * 1024)
                   | (sv & 1023))
        o_v[sl] = ((lax.shift_right_logical(ov, 11) * 1024)
                   | (ov & 1023))
        r_v[sl] = lax.shift_right_logical(rv, 1)
        return carry

    lax.fori_loop(0, W // L, split, 0)


def _emit_out(tmp_v, out_v, iota, g, out16_fn):
    pass


def _e_branch_body(s_hbm, r_hbm, o_hbm, e_hbm, rh_hbm, rt_hbm,
                   out_hbm,
                   s_v, r_v, o_v, sp_v, rp_v, op_v,
                   se_v, oe_v, rh_v, rt_v,
                   tmp_v, out_v, sem):
    wid = lax.axis_index("s") * NC + lax.axis_index("c")
    base0 = wid * CHUNK
    iota = lax.iota(jnp.int32, L)
    for sub in range(NSUB):
        base = base0 + sub * W
        pltpu.sync_copy(s_hbm.at[pl.ds(base, W)], s_v)
        pltpu.sync_copy(r_hbm.at[pl.ds(base, W)], r_v)
        pltpu.sync_copy(o_hbm.at[pl.ds(base, W)], o_v)
        _prep_indices(s_v, r_v, o_v, sp_v, rp_v, op_v)
        copies = [
            pltpu.async_copy(e_hbm.at[s_v], se_v, sem),
            pltpu.async_copy(e_hbm.at[o_v], oe_v, sem),
            pltpu.async_copy(rh_hbm.at[r_v], rh_v, sem),
            pltpu.async_copy(rt_hbm.at[r_v], rt_v, sem),
        ]
        for c in copies:
            c.wait()

        def body(g, carry):
            spg = sp_v[pl.ds(g * L, L)]
            rpg = rp_v[pl.ds(g * L, L)]
            opg = op_v[pl.ds(g * L, L)]
            for bl in range(L):
                b = g * L + bl
                ps = spg[bl]
                pr = rpg[bl]
                po = opg[bl]
                acc = jnp.zeros((L,), jnp.float32)
                for k in range(D // L):
                    sls = pl.ds(ps + k * L, L)
                    slr = pl.ds(pr + k * L, L)
                    slo = pl.ds(po + k * L, L)
                    acc = (acc
                           + se_v[b, sls] * rh_v[b, slr]
                           + oe_v[b, slo] * rt_v[b, slr])
                tmp_v[pl.ds(bl * L, L)] = acc
            out16 = jnp.zeros((L,), jnp.float32)
            row_base = iota * L
            for j in range(L):
                col = plsc.load_gather(tmp_v, [row_base + j])
                out16 = out16 + col
            out_v[pl.ds(g * L, L)] = out16
            return carry

        lax.fori_loop(0, W // L, body, 0)
        pltpu.sync_copy(out_v, out_hbm.at[pl.ds(base, W)])


def _dm_branch_body(s_hbm, r_hbm, o_hbm, edm_hbm, rdm_hbm,
                    out_hbm,
                    s_v, r_v, o_v, sp_v, rp_v, op_v,
                    sdm_v, odm_v, rdm_v,
                    tmp_v, out_v, sem):
    wid = lax.axis_index("s") * NC + lax.axis_index("c")
    base0 = wid * CHUNK
    iota = lax.iota(jnp.int32, L)
    for sub in range(NSUB):
        base = base0 + sub * W
        pltpu.sync_copy(s_hbm.at[pl.ds(base, W)], s_v)
        pltpu.sync_copy(r_hbm.at[pl.ds(base, W)], r_v)
        pltpu.sync_copy(o_hbm.at[pl.ds(base, W)], o_v)
        _prep_indices(s_v, r_v, o_v, sp_v, rp_v, op_v)
        copies = [
            pltpu.async_copy(edm_hbm.at[s_v], sdm_v, sem),
            pltpu.async_copy(edm_hbm.at[o_v], odm_v, sem),
            pltpu.async_copy(rdm_hbm.at[r_v], rdm_v, sem),
        ]
        for c in copies:
            c.wait()

        def body(g, carry):
            spg = sp_v[pl.ds(g * L, L)]
            rpg = rp_v[pl.ds(g * L, L)]
            opg = op_v[pl.ds(g * L, L)]
            for bl in range(L):
                b = g * L + bl
                ps = spg[bl]
                pr = rpg[bl]
                po = opg[bl]
                acc = jnp.zeros((L,), jnp.float32)
                for k in range(D // L):
                    sls = pl.ds(ps + k * L, L)
                    slr = pl.ds(pr + k * L, L)
                    slo = pl.ds(po + k * L, L)
                    acc = (acc
                           + sdm_v[b, sls] * rdm_v[b, slr] * odm_v[b, slo])
                tmp_v[pl.ds(bl * L, L)] = acc
            out16 = jnp.zeros((L,), jnp.float32)
            row_base = iota * L
            for j in range(L):
                col = plsc.load_gather(tmp_v, [row_base + j])
                out16 = out16 + col
            out_v[pl.ds(g * L, L)] = out16
            return carry

        lax.fori_loop(0, W // L, body, 0)
        pltpu.sync_copy(out_v, out_hbm.at[pl.ds(base, W)])


def _sc_kernel(body, n_rows):
    return pl.kernel(
        body,
        out_type=jax.ShapeDtypeStruct((BATCH,), jnp.float32),
        mesh=plsc.VectorSubcoreMesh(core_axis_name="c", subcore_axis_name="s"),
        compiler_params=pltpu.CompilerParams(needs_layout_passes=False),
        scratch_types=(
            [pltpu.VMEM((W,), jnp.int32)] * 6
            + [pltpu.VMEM((W, DP), jnp.float32)] * n_rows
            + [pltpu.VMEM((L * L,), jnp.float32),
               pltpu.VMEM((W,), jnp.float32),
               pltpu.SemaphoreType.DMA]
        ),
    )


@jax.jit
def kernel(s, r, o, E_DM, R_DM, E, R_head, R_tail):
    si = s.astype(jnp.int32)
    ri = r.astype(jnp.int32)
    oi = o.astype(jnp.int32)
    rh2 = R_head.reshape(-1, DP)
    rt2 = R_tail.reshape(-1, DP)
    rdm2 = R_DM.reshape(-1, DP)
    e2 = _transpose_table(E)
    part_a = _sc_kernel(_e_branch_body, 4)(si, ri, oi, e2, rh2, rt2)
    edm2 = _transpose_table(E_DM)
    part_b = _sc_kernel(_dm_branch_body, 3)(si, ri, oi, edm2, rdm2)
    return part_a + part_b


# speedup vs baseline: 1.1600x; 1.1600x over previous
import functools

import jax
import jax.numpy as jnp
from jax import lax
from jax.experimental import pallas as pl
from jax.experimental.pallas import tpu as pltpu
from jax.experimental.pallas import tpu_sc as plsc

BATCH = 16384
NE = 1000000
NR = 1000
D = 64
DP = 128
L = 16
NC = 2
NS = 16
NW = NC * NS
CHUNK = BATCH // NW
W = 128
NSUB = CHUNK // W

BN = 1024
NBLK = pl.cdiv(NE, BN)
NOUT = pl.cdiv(NE, 8 * BN)
NE2 = NOUT * 4 * BN


def _xpose_body(e_ref, d_ref, e2_ref, edm2_ref):
    ident = (lax.broadcasted_iota(jnp.int32, (4 * D, 4 * D), 0)
             == lax.broadcasted_iota(jnp.int32, (4 * D, 4 * D), 1)
             ).astype(jnp.float32)
    dn = (((0,), (0,)), ((), ()))
    hi = lax.Precision.DEFAULT

    def xpose4(x):
        x4 = jnp.concatenate(
            [x[:, kk * BN:(kk + 1) * BN] for kk in range(4)], axis=0)
        return lax.dot_general(x4, ident, dn,
                               preferred_element_type=jnp.float32,
                               precision=hi)

    xe = e_ref[...]
    xd = d_ref[...]
    for h in range(2):
        ye = xpose4(xe[:, 4 * h * BN:(4 * h + 4) * BN])
        yd = xpose4(xd[:, 4 * h * BN:(4 * h + 4) * BN])
        for j in range(4):
            k = 4 * h + j
            half = (k & 1) * D
            row0 = (k >> 1) * BN
            e2_ref[row0:row0 + BN, half:half + D] = ye[:, j * D:(j + 1) * D]
            edm2_ref[row0:row0 + BN, half:half + D] = yd[:, j * D:(j + 1) * D]


def _transpose_tables(E, E_DM):
    in_spec = pl.BlockSpec((D, 8 * BN), lambda i: (0, i))
    out_spec = pl.BlockSpec((4 * BN, DP), lambda i: (i, 0))
    return pl.pallas_call(
        _xpose_body,
        grid=(NOUT,),
        in_specs=[in_spec, in_spec],
        out_specs=[out_spec, out_spec],
        out_shape=[
            jax.ShapeDtypeStruct((NE2, DP), jnp.float32),
            jax.ShapeDtypeStruct((NE2, DP), jnp.float32),
        ],
        compiler_params=pltpu.CompilerParams(
            fuse_transposed_lhs_in_matmul=True),
    )(E.T, E_DM.T)


def _dme_body(s_hbm, r_hbm, o_hbm, edm_hbm, rdm_hbm, e_hbm, rh_hbm, rt_hbm,
              out_hbm,
              s_v, r_v, o_v, sp_v, rp_v, op_v,
              se_v, oe_v, sdm_v, odm_v, rh_v, rt_v, rdm_v,
              tmp_v, out_v, sem):
    wid = lax.axis_index("s") * NC + lax.axis_index("c")
    base0 = wid * CHUNK
    iota = lax.iota(jnp.int32, L)
    for sub in range(NSUB):
        base = base0 + sub * W
        pltpu.sync_copy(s_hbm.at[pl.ds(base, W)], s_v)
        pltpu.sync_copy(r_hbm.at[pl.ds(base, W)], r_v)
        pltpu.sync_copy(o_hbm.at[pl.ds(base, W)], o_v)

        def split(g, carry):
            sl = pl.ds(g * L, L)
            sv = s_v[sl]
            rv = r_v[sl]
            ov = o_v[sl]
            sp_v[sl] = (lax.shift_right_logical(sv, 10) & 1) * D
            op_v[sl] = (lax.shift_right_logical(ov, 10) & 1) * D
            rp_v[sl] = (rv & 1) * D
            s_v[sl] = ((lax.shift_right_logical(sv, 11) * 1024)
                       | (sv & 1023))
            o_v[sl] = ((lax.shift_right_logical(ov, 11) * 1024)
                       | (ov & 1023))
            r_v[sl] = lax.shift_right_logical(rv, 1)
            return carry

        lax.fori_loop(0, W // L, split, 0)

        copies = [
            pltpu.async_copy(e_hbm.at[s_v], se_v, sem),
            pltpu.async_copy(e_hbm.at[o_v], oe_v, sem),
            pltpu.async_copy(edm_hbm.at[s_v], sdm_v, sem),
            pltpu.async_copy(edm_hbm.at[o_v], odm_v, sem),
            pltpu.async_copy(rh_hbm.at[r_v], rh_v, sem),
            pltpu.async_copy(rt_hbm.at[r_v], rt_v, sem),
            pltpu.async_copy(rdm_hbm.at[r_v], rdm_v, sem),
        ]
        for c in copies:
            c.wait()

        def body(g, carry):
            spg = sp_v[pl.ds(g * L, L)]
            rpg = rp_v[pl.ds(g * L, L)]
            opg = op_v[pl.ds(g * L, L)]
            for bl in range(L):
                b = g * L + bl
                ps = spg[bl]
                pr = rpg[bl]
                po = opg[bl]
                acc = jnp.zeros((L,), jnp.float32)
                for k in range(D // L):
                    sls = pl.ds(ps + k * L, L)
                    slr = pl.ds(pr + k * L, L)
                    slo = pl.ds(po + k * L, L)
                    acc = (acc
                           + se_v[b, sls] * rh_v[b, slr]
                           + oe_v[b, slo] * rt_v[b, slr]
                           + sdm_v[b, sls] * rdm_v[b, slr] * odm_v[b, slo])
                tmp_v[pl.ds(bl * L, L)] = acc
            out16 = jnp.zeros((L,), jnp.float32)
            row_base = iota * L
            for j in range(L):
                col = plsc.load_gather(tmp_v, [row_base + j])
                out16 = out16 + col
            out_v[pl.ds(g * L, L)] = out16
            return carry

        lax.fori_loop(0, W // L, body, 0)
        pltpu.sync_copy(out_v, out_hbm.at[pl.ds(base, W)])


@jax.jit
def kernel(s, r, o, E_DM, R_DM, E, R_head, R_tail):
    si = s.astype(jnp.int32)
    ri = r.astype(jnp.int32)
    oi = o.astype(jnp.int32)
    e2, edm2 = _transpose_tables(E, E_DM)
    rh2 = R_head.reshape(-1, DP)
    rt2 = R_tail.reshape(-1, DP)
    rdm2 = R_DM.reshape(-1, DP)
    run = pl.kernel(
        _dme_body,
        out_type=jax.ShapeDtypeStruct((BATCH,), jnp.float32),
        mesh=plsc.VectorSubcoreMesh(core_axis_name="c", subcore_axis_name="s"),
        compiler_params=pltpu.CompilerParams(needs_layout_passes=False),
        scratch_types=[
            pltpu.VMEM((W,), jnp.int32),
            pltpu.VMEM((W,), jnp.int32),
            pltpu.VMEM((W,), jnp.int32),
            pltpu.VMEM((W,), jnp.int32),
            pltpu.VMEM((W,), jnp.int32),
            pltpu.VMEM((W,), jnp.int32),
            pltpu.VMEM((W, DP), jnp.float32),
            pltpu.VMEM((W, DP), jnp.float32),
            pltpu.VMEM((W, DP), jnp.float32),
            pltpu.VMEM((W, DP), jnp.float32),
            pltpu.VMEM((W, DP), jnp.float32),
            pltpu.VMEM((W, DP), jnp.float32),
            pltpu.VMEM((W, DP), jnp.float32),
            pltpu.VMEM((L * L,), jnp.float32),
            pltpu.VMEM((W,), jnp.float32),
            pltpu.SemaphoreType.DMA,
        ],
    )
    return run(si, ri, oi, edm2, rdm2, e2, rh2, rt2)
